# baseline (speedup 1.0000x reference)
import math

import jax
import jax.numpy as jnp
from jax import lax
from jax.experimental import pallas as pl
from jax.experimental.pallas import tpu as pltpu
from jax.experimental.pallas import tpu_sc as plsc

VOCAB = 100000
EMB = 128
SEQ = 50
SCALE = math.sqrt(EMB)

NC, NS, L = 2, 16, 16
NW = NC * NS

NTOK = 4096
NSPLIT = 2
NTOK_P = NTOK // NSPLIT
T_PER_W = NTOK_P // NW
TPC = 2
CHUNK = TPC * SEQ
N_CHUNKS = T_PER_W // TPC
NBUF = 6


def _emb_body(tok_hbm, table_hbm, out_hbm, idx_all, rows_v, *sems):
    gsem = sems[:NBUF]
    ssem = sems[NBUF:]
    wid = lax.axis_index("c") * NS + lax.axis_index("s")
    tbase = wid * T_PER_W

    pltpu.sync_copy(tok_hbm.at[wid], idx_all)

    def gather(j, b):
        return pltpu.async_copy(table_hbm.at[idx_all.at[j]], rows_v.at[b],
                                gsem[b])

    def store(j, b):
        return [
            pltpu.async_copy(rows_v.at[b, pl.ds(t * SEQ, SEQ)],
                             out_hbm.at[tbase + j * TPC + t],
                             ssem[b * TPC + t])
            for t in range(TPC)
        ]

    def scale(b):
        @plsc.parallel_loop(0, CHUNK, step=1, unroll=4)
        def srow(i):
            for c in range(EMB // L):
                sl = (b, i, pl.ds(c * L, L))
                rows_v[sl] = rows_v[sl] * SCALE

    gd, sd = {}, {}
    for j in range(min(NBUF - 1, N_CHUNKS)):
        gd[j] = gather(j, j % NBUF)
    for j in range(N_CHUNKS):
        b = j % NBUF
        jn = j + NBUF - 1
        if jn < N_CHUNKS:
            if jn - NBUF >= 0:
                for h in sd[jn - NBUF]:
                    h.wait()
            gd[jn] = gather(jn, jn % NBUF)
        gd[j].wait()
        scale(b)
        sd[j] = store(j, b)
    for j in range(max(0, N_CHUNKS - NBUF), N_CHUNKS):
        for h in sd[j]:
            h.wait()


def _emb(tokens_grid, table):
    mesh = plsc.VectorSubcoreMesh(core_axis_name="c", subcore_axis_name="s")
    f = pl.kernel(
        _emb_body,
        out_type=jax.ShapeDtypeStruct((NTOK_P, SEQ, EMB), jnp.float32),
        mesh=mesh,
        scratch_types=[
            pltpu.VMEM((N_CHUNKS, CHUNK), jnp.int32),
            pltpu.VMEM((NBUF, CHUNK, EMB), jnp.float32),
        ] + [pltpu.SemaphoreType.DMA] * (NBUF + NBUF * TPC),
    )
    return f(tokens_grid, table)


def kernel(tokens, table):
    parts = []
    for p in range(NSPLIT):
        tp = lax.slice_in_dim(tokens, p * NTOK_P, (p + 1) * NTOK_P, axis=0)
        tok = tp.reshape(NW, N_CHUNKS, CHUNK).astype(jnp.int32)
        parts.append(_emb(tok, table))
    return jnp.concatenate(parts, axis=0)


# speedup vs baseline: 1.5859x; 1.5859x over previous
import math

import jax
import jax.numpy as jnp
from jax import lax
from jax.experimental import pallas as pl
from jax.experimental.pallas import tpu as pltpu
from jax.experimental.pallas import tpu_sc as plsc

VOCAB = 100000
EMB = 128
SEQ = 50
SCALE = math.sqrt(EMB)

NC, NS, L = 2, 16, 16
NW = NC * NS

NTOK = 4096
T_PER_W = NTOK // NW
TPC = 2
CHUNK = TPC * SEQ
N_CHUNKS = T_PER_W // TPC
NBUF = 6


def _emb_body(tok_hbm, table_hbm, out_hbm, idx_all, rows_v, *sems):
    gsem = sems[:NBUF]
    ssem = sems[NBUF:]
    wid = lax.axis_index("s") * NC + lax.axis_index("c")
    tbase = wid * T_PER_W

    pltpu.sync_copy(tok_hbm.at[wid], idx_all)

    def gather(j, b):
        return pltpu.async_copy(table_hbm.at[idx_all.at[j]], rows_v.at[b],
                                gsem[b])

    def store(j, b):
        return [
            pltpu.async_copy(rows_v.at[b, pl.ds(t * SEQ, SEQ)],
                             out_hbm.at[tbase + j * TPC + t],
                             ssem[b * TPC + t])
            for t in range(TPC)
        ]

    def scale(b):
        @plsc.parallel_loop(0, CHUNK, step=1, unroll=4)
        def srow(i):
            for c in range(EMB // L):
                sl = (b, i, pl.ds(c * L, L))
                rows_v[sl] = rows_v[sl] * SCALE

    gd, sd = {}, {}
    for j in range(min(NBUF - 1, N_CHUNKS)):
        gd[j] = gather(j, j % NBUF)
    for j in range(N_CHUNKS):
        b = j % NBUF
        jn = j + NBUF - 1
        if jn < N_CHUNKS:
            if jn - NBUF >= 0:
                for h in sd[jn - NBUF]:
                    h.wait()
            gd[jn] = gather(jn, jn % NBUF)
        gd[j].wait()
        scale(b)
        sd[j] = store(j, b)
    for j in range(max(0, N_CHUNKS - NBUF), N_CHUNKS):
        for h in sd[j]:
            h.wait()


@jax.jit
def _emb(tokens_grid, table):
    mesh = plsc.VectorSubcoreMesh(core_axis_name="c", subcore_axis_name="s")
    f = pl.kernel(
        _emb_body,
        out_type=jax.ShapeDtypeStruct((NTOK, SEQ, EMB), jnp.float32),
        compiler_params=pltpu.CompilerParams(use_tc_tiling_on_sc=True),
        mesh=mesh,
        scratch_types=[
            pltpu.VMEM((N_CHUNKS, CHUNK), jnp.int32),
            pltpu.VMEM((NBUF, CHUNK, EMB), jnp.float32),
        ] + [pltpu.SemaphoreType.DMA] * (NBUF + NBUF * TPC),
    )
    return f(tokens_grid, table)


def kernel(tokens, table):
    tok = tokens.reshape(NW, N_CHUNKS, CHUNK).astype(jnp.int32)
    return _emb(tok, table)
